# baseline (speedup 1.0000x reference)
import functools

import jax
import jax.numpy as jnp
from jax import lax
from jax.experimental import pallas as pl
from jax.experimental.pallas import tpu as pltpu

_WIN = 4
_W2 = 2 * _WIN


def _fused_body(Tx, x_ref, wp1_hbm, wp2_ref, win_hbm, wout_hbm, ctx_hbm,
                h_ref, attn_ref,
                wp1_v, win_v, wout_v, ctx_v,
                sem_wp1, sem_win, sem_wout, sem_ctx):
    B, dim = x_ref.shape
    cp_wp1 = pltpu.make_async_copy(wp1_hbm, wp1_v, sem_wp1)
    nchunk = 4
    chunk = dim // nchunk
    cp_win = [pltpu.make_async_copy(win_hbm, win_v, sem_win)]
    cp_wout = [
        pltpu.make_async_copy(
            wout_hbm.at[pl.ds(i * chunk, chunk), :],
            wout_v.at[pl.ds(i * chunk, chunk), :], sem_wout[i])
        for i in range(nchunk)
    ]
    cp_wp1.start()
    for cp in cp_win:
        cp.start()

    x = x_ref[...]
    cp_wp1.wait()
    p = lax.dot_general(x, wp1_v[...], (((1,), (1,)), ((), ())))
    p = jnp.tanh(p)
    w8 = jnp.broadcast_to(wp2_ref[...], (_W2, wp2_ref.shape[1]))
    p = lax.dot_general(p, w8, (((1,), (1,)), ((), ())))[:, :1]
    pt = Tx * jax.nn.sigmoid(p)
    bl = (pt - _WIN).astype(jnp.int32)
    j = lax.broadcasted_iota(jnp.int32, (B, _W2), 1)
    idx = jnp.clip(bl + j, 0, Tx - 1)
    start = jnp.clip(bl[:, 0], 0, Tx - _W2)
    sa = jnp.clip((start // 8) * 8, 0, Tx - 2 * _W2)
    rel = idx - sa[:, None]
    onehot = (rel[:, :, None]
              == lax.broadcasted_iota(jnp.int32, (B, _W2, 2 * _W2), 2)
              ).astype(jnp.float32)

    ctx_cps = []
    for b in range(B):
        cp = pltpu.make_async_copy(
            ctx_hbm.at[b, pl.ds(pl.multiple_of(sa[b], 8), 2 * _W2), :],
            ctx_v.at[b], sem_ctx)
        cp.start()
        ctx_cps.append(cp)

    for cp in cp_wout:
        cp.start()

    for cp in cp_win:
        cp.wait()
    t = lax.dot_general(x, win_v[...], (((1,), (1,)), ((), ())))

    for cp in ctx_cps:
        cp.wait()
    w = ctx_v[...]

    rows = []
    for b in range(B):
        tb8 = jnp.broadcast_to(t[b : b + 1, :], (_W2, dim))
        rows.append(
            lax.dot_general(tb8, w[b], (((1,), (1,)), ((), ())))[:1, :]
        )
    scores16 = jnp.concatenate(rows, axis=0)
    logits = jnp.sum(onehot * scores16[:, None, :], axis=2)
    m = jnp.max(logits, axis=1, keepdims=True)
    e = jnp.exp(logits - m)
    a = e / jnp.sum(e, axis=1, keepdims=True)
    attn_ref[...] = a
    a16 = jnp.sum(onehot * a[:, :, None], axis=1)
    wcs = [
        lax.dot_general(a16[b : b + 1, :], w[b], (((1,), (0,)), ((), ())))
        for b in range(B)
    ]
    wc = jnp.concatenate(wcs, axis=0)

    for i, cp in enumerate(cp_wout):
        cp.wait()
        wo = wout_v[i * chunk : (i + 1) * chunk, :]
        h = lax.dot_general(wc, wo[:, :dim], (((1,), (1,)), ((), ())))
        h = h + lax.dot_general(x, wo[:, dim:], (((1,), (1,)), ((), ())))
        h_ref[:, i * chunk : (i + 1) * chunk] = jnp.tanh(h)


def kernel(input, context, W_in, W_out, W_p1, W_p2):
    B, Tx, dim = context.shape
    hbm = pl.BlockSpec(memory_space=pl.ANY)
    return pl.pallas_call(
        functools.partial(_fused_body, Tx),
        in_specs=[
            pl.BlockSpec(memory_space=pltpu.VMEM),
            hbm,
            pl.BlockSpec(memory_space=pltpu.VMEM),
            hbm,
            hbm,
            hbm,
        ],
        out_shape=[
            jax.ShapeDtypeStruct((B, dim), jnp.float32),
            jax.ShapeDtypeStruct((B, _W2), jnp.float32),
        ],
        scratch_shapes=[
            pltpu.VMEM((dim // 2, dim), jnp.float32),
            pltpu.VMEM((dim, dim), jnp.float32),
            pltpu.VMEM((dim, 2 * dim), jnp.float32),
            pltpu.VMEM((B, 2 * _W2, dim), jnp.float32),
            pltpu.SemaphoreType.DMA,
            pltpu.SemaphoreType.DMA,
            [pltpu.SemaphoreType.DMA] * 4,
            pltpu.SemaphoreType.DMA,
        ],
    )(input, W_p1, W_p2, W_in, W_out, context)


# speedup vs baseline: 1.0570x; 1.0570x over previous
import functools

import jax
import jax.numpy as jnp
from jax import lax
from jax.experimental import pallas as pl
from jax.experimental.pallas import tpu as pltpu

_WIN = 4
_W2 = 2 * _WIN


def _fused_body(Tx, x_ref, wp1_hbm, wp2_ref, win_hbm, wout_hbm, ctx_hbm,
                h_ref, attn_ref,
                wp1_v, win_v, wout_v, ctx_v,
                sem_wp1, sem_win, sem_wout, sem_ctx):
    B, dim = x_ref.shape
    cp_wp1 = pltpu.make_async_copy(wp1_hbm, wp1_v, sem_wp1)
    nchunk = 2
    chunk = dim // nchunk
    cp_win = [pltpu.make_async_copy(win_hbm, win_v, sem_win)]
    cp_wout = [
        pltpu.make_async_copy(
            wout_hbm.at[pl.ds(i * chunk, chunk), :],
            wout_v.at[pl.ds(i * chunk, chunk), :], sem_wout[i])
        for i in range(nchunk)
    ]
    cp_wp1.start()
    for cp in cp_win:
        cp.start()

    x = x_ref[...]
    cp_wp1.wait()
    p = lax.dot_general(x, wp1_v[...], (((1,), (1,)), ((), ())))
    p = jnp.tanh(p)
    w8 = jnp.broadcast_to(wp2_ref[...], (_W2, wp2_ref.shape[1]))
    p = lax.dot_general(p, w8, (((1,), (1,)), ((), ())))[:, :1]
    pt = Tx * jax.nn.sigmoid(p)
    bl = (pt - _WIN).astype(jnp.int32)
    j = lax.broadcasted_iota(jnp.int32, (B, _W2), 1)
    idx = jnp.clip(bl + j, 0, Tx - 1)
    start = jnp.clip(bl[:, 0], 0, Tx - _W2)
    sa = jnp.clip((start // 8) * 8, 0, Tx - 2 * _W2)
    rel = idx - sa[:, None]
    onehot = (rel[:, :, None]
              == lax.broadcasted_iota(jnp.int32, (B, _W2, 2 * _W2), 2)
              ).astype(jnp.float32)

    ctx_cps = []
    for b in range(B):
        cp = pltpu.make_async_copy(
            ctx_hbm.at[b, pl.ds(pl.multiple_of(sa[b], 8), 2 * _W2), :],
            ctx_v.at[b], sem_ctx)
        cp.start()
        ctx_cps.append(cp)

    for cp in cp_wout:
        cp.start()

    for cp in cp_win:
        cp.wait()
    t = lax.dot_general(x, win_v[...], (((1,), (1,)), ((), ())))

    for cp in ctx_cps:
        cp.wait()
    w = ctx_v[...]

    rows = []
    for b in range(B):
        tb8 = jnp.broadcast_to(t[b : b + 1, :], (_W2, dim))
        rows.append(
            lax.dot_general(tb8, w[b], (((1,), (1,)), ((), ())))[:1, :]
        )
    scores16 = jnp.concatenate(rows, axis=0)
    logits = jnp.sum(onehot * scores16[:, None, :], axis=2)
    m = jnp.max(logits, axis=1, keepdims=True)
    e = jnp.exp(logits - m)
    a = e / jnp.sum(e, axis=1, keepdims=True)
    attn_ref[...] = a
    a16 = jnp.sum(onehot * a[:, :, None], axis=1)
    wcs = [
        lax.dot_general(a16[b : b + 1, :], w[b], (((1,), (0,)), ((), ())))
        for b in range(B)
    ]
    wc = jnp.concatenate(wcs, axis=0)

    for i, cp in enumerate(cp_wout):
        cp.wait()
        wo = wout_v[i * chunk : (i + 1) * chunk, :]
        h = lax.dot_general(wc, wo[:, :dim], (((1,), (1,)), ((), ())))
        h = h + lax.dot_general(x, wo[:, dim:], (((1,), (1,)), ((), ())))
        h_ref[:, i * chunk : (i + 1) * chunk] = jnp.tanh(h)


def kernel(input, context, W_in, W_out, W_p1, W_p2):
    B, Tx, dim = context.shape
    hbm = pl.BlockSpec(memory_space=pl.ANY)
    return pl.pallas_call(
        functools.partial(_fused_body, Tx),
        in_specs=[
            pl.BlockSpec(memory_space=pltpu.VMEM),
            hbm,
            pl.BlockSpec(memory_space=pltpu.VMEM),
            hbm,
            hbm,
            hbm,
        ],
        out_shape=[
            jax.ShapeDtypeStruct((B, dim), jnp.float32),
            jax.ShapeDtypeStruct((B, _W2), jnp.float32),
        ],
        scratch_shapes=[
            pltpu.VMEM((dim // 2, dim), jnp.float32),
            pltpu.VMEM((dim, dim), jnp.float32),
            pltpu.VMEM((dim, 2 * dim), jnp.float32),
            pltpu.VMEM((B, 2 * _W2, dim), jnp.float32),
            pltpu.SemaphoreType.DMA,
            pltpu.SemaphoreType.DMA,
            [pltpu.SemaphoreType.DMA] * 2,
            pltpu.SemaphoreType.DMA,
        ],
    )(input, W_p1, W_p2, W_in, W_out, context)
